# baseline (speedup 1.0000x reference)
import jax
import jax.numpy as jnp
from jax import lax
from jax.experimental import pallas as pl
from jax.experimental.pallas import tpu as pltpu
from jax.experimental.pallas import tpu_sc as plsc

N = 10000
E = 320000
D = 128
NC = 2
NS = 16
NW = NC * NS
EPT = E // NW
K = 128
EPT2 = 10240
NCHUNK = EPT2 // K
NPAD = 10240
EPS = NPAD // NS


def _mesh():
    return plsc.VectorSubcoreMesh(core_axis_name="c", subcore_axis_name="s")


def _deg_kernel(sd_hbm, zero_hbm, degpart_hbm, sdl, ones_v, shared_deg):
    cid = lax.axis_index("c")
    sid = lax.axis_index("s")
    wid = sid * NC + cid
    pltpu.sync_copy(zero_hbm.at[pl.ds(sid * EPS, EPS)],
                    shared_deg.at[pl.ds(sid * EPS, EPS)])
    pltpu.sync_copy(sd_hbm.at[wid], sdl)
    for i in range(K // 16):
        ones_v[pl.ds(i * 16, 16)] = jnp.full((16,), 1.0, jnp.float32)
    plsc.subcore_barrier()

    def body(c, carry):
        pltpu.sync_copy(ones_v, shared_deg.at[sdl.at[c, 1]], add=True)
        return carry

    lax.fori_loop(0, NCHUNK, body, 0)
    plsc.subcore_barrier()
    pltpu.sync_copy(shared_deg.at[pl.ds(sid * EPS, EPS)],
                    degpart_hbm.at[pl.ds(cid * NPAD + sid * EPS, EPS)])


def _deg_pass(sd, zeros_pad):
    k = pl.kernel(
        _deg_kernel,
        out_type=jax.ShapeDtypeStruct((NC * NPAD,), jnp.float32),
        mesh=_mesh(),
        compiler_params=pltpu.CompilerParams(needs_layout_passes=False),
        scratch_types=[
            pltpu.VMEM((NCHUNK, 2, K), jnp.int32),
            pltpu.VMEM((K,), jnp.float32),
            pltpu.VMEM_SHARED((NPAD,), jnp.float32),
        ],
    )
    return k(sd, zeros_pad)


PAIRS = NCHUNK // 2


def _edge_kernel(x_hbm, u_hbm, v_hbm, sd_hbm, zh_hbm, ze_hbm,
                 hpart_hbm, epart_hbm,
                 sdbuf, rows2, ex2, scal, shared_h, shared_e,
                 sem_ra, sem_rb, sem_ua, sem_va, sem_ub, sem_vb,
                 sem_ea, sem_eb, sem_ha, sem_hb):
    cid = lax.axis_index("c")
    sid = lax.axis_index("s")
    wid = sid * NC + cid
    pltpu.sync_copy(zh_hbm.at[pl.ds(sid * EPS, EPS)],
                    shared_h.at[pl.ds(sid * EPS, EPS)])
    pltpu.sync_copy(ze_hbm.at[pl.ds(sid * EPS, EPS)],
                    shared_e.at[pl.ds(sid * EPS, EPS)])

    z16 = jnp.zeros((16,), jnp.float32)

    def zr(j, carry):
        for db in range(D // 16):
            rows2[0, j, pl.ds(db * 16, 16)] = z16
            rows2[1, j, pl.ds(db * 16, 16)] = z16
        return carry

    lax.fori_loop(0, K, zr, 0)
    for i in range(K // 16):
        ex2[0, pl.ds(i * 16, 16)] = z16
        ex2[1, pl.ds(i * 16, 16)] = z16
    plsc.subcore_barrier()

    pltpu.sync_copy(sd_hbm.at[wid, pl.ds(0, 2)], sdbuf)
    pltpu.async_copy(ex2.at[0], shared_e.at[sdbuf.at[0, 1]], sem_ea, add=True)
    pltpu.async_copy(ex2.at[1], shared_e.at[sdbuf.at[1, 1]], sem_eb, add=True)
    pltpu.async_copy(rows2.at[0], shared_h.at[sdbuf.at[0, 1]], sem_ha, add=True)
    pltpu.async_copy(rows2.at[1], shared_h.at[sdbuf.at[1, 1]], sem_hb, add=True)

    def process_slot(slot, du, dv, dr, se, sh):
        du.wait()
        dv.wait()
        for i in range(K // 16):
            e = scal[slot, 0, pl.ds(i * 16, 16)] + scal[slot, 1, pl.ds(i * 16, 16)]
            e = jnp.where(e >= 0.0, e, e * 0.2)
            ex2[slot, pl.ds(i * 16, 16)] = jnp.exp(e)
        pltpu.async_copy(ex2.at[slot], shared_e.at[sdbuf.at[slot, 1]], se,
                         add=True)
        dr.wait()

        def scale(kk, inner):
            s = plsc.load_gather(ex2.at[slot],
                                 [jnp.zeros((16,), jnp.int32) + kk])
            for db in range(D // 16):
                rows2[slot, kk, pl.ds(db * 16, 16)] = (
                    rows2[slot, kk, pl.ds(db * 16, 16)] * s)
            return inner

        lax.fori_loop(0, K, scale, 0)
        pltpu.async_copy(rows2.at[slot], shared_h.at[sdbuf.at[slot, 1]], sh,
                         add=True)

    def drain_all():
        pltpu.make_async_copy(ex2.at[0], shared_e.at[sdbuf.at[0, 1]], sem_ea).wait()
        pltpu.make_async_copy(ex2.at[1], shared_e.at[sdbuf.at[1, 1]], sem_eb).wait()
        pltpu.make_async_copy(rows2.at[0], shared_h.at[sdbuf.at[0, 1]], sem_ha).wait()
        pltpu.make_async_copy(rows2.at[1], shared_h.at[sdbuf.at[1, 1]], sem_hb).wait()

    def body(i, carry):
        drain_all()
        pltpu.sync_copy(sd_hbm.at[wid, pl.ds(2 * i, 2)], sdbuf)
        dra = pltpu.async_copy(x_hbm.at[sdbuf.at[0, 0]], rows2.at[0], sem_ra)
        drb = pltpu.async_copy(x_hbm.at[sdbuf.at[1, 0]], rows2.at[1], sem_rb)
        dua = pltpu.async_copy(u_hbm.at[sdbuf.at[0, 0]], scal.at[0, 0], sem_ua)
        dva = pltpu.async_copy(v_hbm.at[sdbuf.at[0, 1]], scal.at[0, 1], sem_va)
        dub = pltpu.async_copy(u_hbm.at[sdbuf.at[1, 0]], scal.at[1, 0], sem_ub)
        dvb = pltpu.async_copy(v_hbm.at[sdbuf.at[1, 1]], scal.at[1, 1], sem_vb)
        process_slot(0, dua, dva, dra, sem_ea, sem_ha)
        process_slot(1, dub, dvb, drb, sem_eb, sem_hb)
        return carry

    lax.fori_loop(0, PAIRS, body, 0)
    drain_all()
    plsc.subcore_barrier()
    pltpu.sync_copy(shared_h.at[pl.ds(sid * EPS, EPS)],
                    hpart_hbm.at[pl.ds(cid * NPAD + sid * EPS, EPS)])
    pltpu.sync_copy(shared_e.at[pl.ds(sid * EPS, EPS)],
                    epart_hbm.at[pl.ds(cid * NPAD + sid * EPS, EPS)])


def _edge_pass(x, u, v, sd, zeros_h, zeros_pad):
    k = pl.kernel(
        _edge_kernel,
        out_type=(jax.ShapeDtypeStruct((NC * NPAD, D), jnp.float32),
                  jax.ShapeDtypeStruct((NC * NPAD,), jnp.float32)),
        mesh=_mesh(),
        compiler_params=pltpu.CompilerParams(needs_layout_passes=False),
        scratch_types=[
            pltpu.VMEM((2, 2, K), jnp.int32),
            pltpu.VMEM((2, K, D), jnp.float32),
            pltpu.VMEM((2, K), jnp.float32),
            pltpu.VMEM((2, 2, K), jnp.float32),
            pltpu.VMEM_SHARED((NPAD, D), jnp.float32),
            pltpu.VMEM_SHARED((NPAD,), jnp.float32),
        ] + [pltpu.SemaphoreType.DMA] * 10,
    )
    return k(x, u, v, sd, zeros_h, zeros_pad)


_GRID = 10
_RB = N // _GRID


def _prologue_kernel(degpair_ref, feats_ref, wc_ref, norm_ref, u_ref, v_ref):
    deg = degpair_ref[:, 0:1] + degpair_ref[:, 1:2]
    norm = lax.rsqrt(jnp.maximum(deg, 1.0))
    pq = jnp.dot(feats_ref[...], wc_ref[...], preferred_element_type=jnp.float32)
    norm_ref[...] = norm
    u_ref[...] = norm * pq[:, 0:1]
    v_ref[...] = norm * pq[:, 1:2]


def _prologue(degpair, feats, wcat):
    return pl.pallas_call(
        _prologue_kernel,
        grid=(_GRID,),
        in_specs=[
            pl.BlockSpec((_RB, 2), lambda i: (i, 0)),
            pl.BlockSpec((_RB, D), lambda i: (i, 0)),
            pl.BlockSpec((D, 2), lambda i: (0, 0)),
        ],
        out_specs=[
            pl.BlockSpec((_RB, 1), lambda i: (i, 0)),
            pl.BlockSpec((_RB, 1), lambda i: (i, 0)),
            pl.BlockSpec((_RB, 1), lambda i: (i, 0)),
        ],
        out_shape=[jax.ShapeDtypeStruct((N, 1), jnp.float32)] * 3,
    )(degpair, feats, wcat)


def _epilogue_kernel(hp_ref, ep_ref, y_ref, norm_ref, wc_ref, sc_ref,
                     x_ref, yo_ref, u_ref, v_ref):
    es = ep_ref[:, 0:1] + ep_ref[:, 1:2] + 1e-16
    h = (hp_ref[0] + hp_ref[1]) / es
    x_ref[...] = h
    yo_ref[...] = (y_ref[...] + h) * sc_ref[0, 0]
    norm = norm_ref[...]
    pq = jnp.dot(h, wc_ref[...], preferred_element_type=jnp.float32)
    u_ref[...] = norm * pq[:, 0:1]
    v_ref[...] = norm * pq[:, 1:2]


def _epilogue(hpart, epair, y_prev, norm, wcat, sc):
    return pl.pallas_call(
        _epilogue_kernel,
        grid=(_GRID,),
        in_specs=[
            pl.BlockSpec((NC, _RB, D), lambda i: (0, i, 0)),
            pl.BlockSpec((_RB, 2), lambda i: (i, 0)),
            pl.BlockSpec((_RB, D), lambda i: (i, 0)),
            pl.BlockSpec((_RB, 1), lambda i: (i, 0)),
            pl.BlockSpec((D, 2), lambda i: (0, 0)),
            pl.BlockSpec((1, 1), lambda i: (0, 0)),
        ],
        out_specs=[
            pl.BlockSpec((_RB, D), lambda i: (i, 0)),
            pl.BlockSpec((_RB, D), lambda i: (i, 0)),
            pl.BlockSpec((_RB, 1), lambda i: (i, 0)),
            pl.BlockSpec((_RB, 1), lambda i: (i, 0)),
        ],
        out_shape=[
            jax.ShapeDtypeStruct((N, D), jnp.float32),
            jax.ShapeDtypeStruct((N, D), jnp.float32),
            jax.ShapeDtypeStruct((N, 1), jnp.float32),
            jax.ShapeDtypeStruct((N, 1), jnp.float32),
        ],
    )(hpart, epair, y_prev, norm, wcat, sc)


def kernel(feats, edge_index, order, W_att):
    srcp = jnp.pad(edge_index[0].astype(jnp.int32).reshape(NW, EPT),
                   ((0, 0), (0, EPT2 - EPT)))
    dstp = jnp.pad(edge_index[1].astype(jnp.int32).reshape(NW, EPT),
                   ((0, 0), (0, EPT2 - EPT)), constant_values=N)
    sd = jnp.stack([srcp.reshape(NW, NCHUNK, K),
                    dstp.reshape(NW, NCHUNK, K)], axis=2)
    wcat = W_att[:, 0].reshape(2, D).T
    zeros_pad = jnp.zeros((NPAD,), jnp.float32)
    zeros_h = jnp.zeros((NPAD, D), jnp.float32)

    degpart = _deg_pass(sd, zeros_pad)
    degpair = degpart.reshape(NC, NPAD)[:, :N].T
    norm, u, v = _prologue(degpair, feats, wcat)

    one = jnp.ones((1, 1), jnp.float32)
    last = (1.0 / (order + 1.0)) * one

    x = feats
    y = feats
    for t in range(4):
        upad = jnp.concatenate([u.reshape(N), jnp.zeros(NPAD - N, jnp.float32)])
        vpad = jnp.concatenate([v.reshape(N), jnp.zeros(NPAD - N, jnp.float32)])
        hflat, eflat = _edge_pass(x, upad, vpad, sd, zeros_h, zeros_pad)
        hpart = hflat.reshape(NC, NPAD, D)
        epair = eflat.reshape(NC, NPAD)[:, :N].T
        sc = last if t == 3 else one
        x, y, u, v = _epilogue(hpart, epair, y, norm, wcat, sc)
    return y


# speedup vs baseline: 1.4130x; 1.4130x over previous
import jax
import jax.numpy as jnp
from jax import lax
from jax.experimental import pallas as pl
from jax.experimental.pallas import tpu as pltpu
from jax.experimental.pallas import tpu_sc as plsc

N = 10000
E = 320000
D = 128
NC = 2
NS = 16
NW = NC * NS
EPT = E // NW
K = 80
EPT2 = 10080
NCHUNK = EPT2 // K
NPAD = 10240
EPS = NPAD // NS


def _mesh():
    return plsc.VectorSubcoreMesh(core_axis_name="c", subcore_axis_name="s")


def _deg_kernel(sd_hbm, zero_hbm, degpart_hbm, sdl, ones_v, shared_deg):
    cid = lax.axis_index("c")
    sid = lax.axis_index("s")
    wid = sid * NC + cid
    pltpu.sync_copy(zero_hbm.at[pl.ds(sid * EPS, EPS)],
                    shared_deg.at[pl.ds(sid * EPS, EPS)])
    pltpu.sync_copy(sd_hbm.at[wid], sdl)
    for i in range(K // 16):
        ones_v[pl.ds(i * 16, 16)] = jnp.full((16,), 1.0, jnp.float32)
    plsc.subcore_barrier()

    def body(c, carry):
        pltpu.sync_copy(ones_v, shared_deg.at[sdl.at[c, 1]], add=True)
        return carry

    lax.fori_loop(0, NCHUNK, body, 0)
    plsc.subcore_barrier()
    pltpu.sync_copy(shared_deg.at[pl.ds(sid * EPS, EPS)],
                    degpart_hbm.at[pl.ds(cid * NPAD + sid * EPS, EPS)])


def _deg_pass(sd, zeros_pad):
    k = pl.kernel(
        _deg_kernel,
        out_type=jax.ShapeDtypeStruct((NC * NPAD,), jnp.float32),
        mesh=_mesh(),
        compiler_params=pltpu.CompilerParams(needs_layout_passes=False),
        scratch_types=[
            pltpu.VMEM((NCHUNK, 2, K), jnp.int32),
            pltpu.VMEM((K,), jnp.float32),
            pltpu.VMEM_SHARED((NPAD,), jnp.float32),
        ],
    )
    return k(sd, zeros_pad)


PAIRS = NCHUNK // 2


def _edge_kernel(x_hbm, u_hbm, v_hbm, sd_hbm, zh_hbm, ze_hbm,
                 hpart_hbm, epart_hbm,
                 sdbuf, rows2, ex2, scal, shared_h, shared_e,
                 sem_ra, sem_rb, sem_ua, sem_va, sem_ub, sem_vb,
                 sem_ea, sem_eb, sem_ha, sem_hb):
    cid = lax.axis_index("c")
    sid = lax.axis_index("s")
    wid = sid * NC + cid
    pltpu.sync_copy(zh_hbm.at[pl.ds(sid * EPS, EPS)],
                    shared_h.at[pl.ds(sid * EPS, EPS)])
    pltpu.sync_copy(ze_hbm.at[pl.ds(sid * EPS, EPS)],
                    shared_e.at[pl.ds(sid * EPS, EPS)])

    z16 = jnp.zeros((16,), jnp.float32)

    def zr(j, carry):
        for db in range(D // 16):
            rows2[0, j, pl.ds(db * 16, 16)] = z16
            rows2[1, j, pl.ds(db * 16, 16)] = z16
        return carry

    lax.fori_loop(0, K, zr, 0)
    for i in range(K // 16):
        ex2[0, pl.ds(i * 16, 16)] = z16
        ex2[1, pl.ds(i * 16, 16)] = z16
    plsc.subcore_barrier()

    pltpu.sync_copy(sd_hbm.at[wid, pl.ds(0, 2)], sdbuf)
    pltpu.async_copy(ex2.at[0], shared_e.at[sdbuf.at[0, 1]], sem_ea, add=True)
    pltpu.async_copy(ex2.at[1], shared_e.at[sdbuf.at[1, 1]], sem_eb, add=True)
    pltpu.async_copy(rows2.at[0], shared_h.at[sdbuf.at[0, 1]], sem_ha, add=True)
    pltpu.async_copy(rows2.at[1], shared_h.at[sdbuf.at[1, 1]], sem_hb, add=True)

    def process_slot(slot, du, dv, dr, se, sh):
        du.wait()
        dv.wait()
        for i in range(K // 16):
            e = scal[slot, 0, pl.ds(i * 16, 16)] + scal[slot, 1, pl.ds(i * 16, 16)]
            e = jnp.where(e >= 0.0, e, e * 0.2)
            ex2[slot, pl.ds(i * 16, 16)] = jnp.exp(e)
        pltpu.async_copy(ex2.at[slot], shared_e.at[sdbuf.at[slot, 1]], se,
                         add=True)
        dr.wait()

        def scale(kk, inner):
            s = plsc.load_gather(ex2.at[slot],
                                 [jnp.zeros((16,), jnp.int32) + kk])
            for db in range(D // 16):
                rows2[slot, kk, pl.ds(db * 16, 16)] = (
                    rows2[slot, kk, pl.ds(db * 16, 16)] * s)
            return inner

        lax.fori_loop(0, K, scale, 0)
        pltpu.async_copy(rows2.at[slot], shared_h.at[sdbuf.at[slot, 1]], sh,
                         add=True)

    def drain_all():
        pltpu.make_async_copy(ex2.at[0], shared_e.at[sdbuf.at[0, 1]], sem_ea).wait()
        pltpu.make_async_copy(ex2.at[1], shared_e.at[sdbuf.at[1, 1]], sem_eb).wait()
        pltpu.make_async_copy(rows2.at[0], shared_h.at[sdbuf.at[0, 1]], sem_ha).wait()
        pltpu.make_async_copy(rows2.at[1], shared_h.at[sdbuf.at[1, 1]], sem_hb).wait()

    def body(i, carry):
        drain_all()
        pltpu.sync_copy(sd_hbm.at[wid, pl.ds(2 * i, 2)], sdbuf)
        dra = pltpu.async_copy(x_hbm.at[sdbuf.at[0, 0]], rows2.at[0], sem_ra)
        drb = pltpu.async_copy(x_hbm.at[sdbuf.at[1, 0]], rows2.at[1], sem_rb)
        dua = pltpu.async_copy(u_hbm.at[sdbuf.at[0, 0]], scal.at[0, 0], sem_ua)
        dva = pltpu.async_copy(v_hbm.at[sdbuf.at[0, 1]], scal.at[0, 1], sem_va)
        dub = pltpu.async_copy(u_hbm.at[sdbuf.at[1, 0]], scal.at[1, 0], sem_ub)
        dvb = pltpu.async_copy(v_hbm.at[sdbuf.at[1, 1]], scal.at[1, 1], sem_vb)
        process_slot(0, dua, dva, dra, sem_ea, sem_ha)
        process_slot(1, dub, dvb, drb, sem_eb, sem_hb)
        return carry

    lax.fori_loop(0, PAIRS, body, 0)
    drain_all()
    plsc.subcore_barrier()
    pltpu.sync_copy(shared_h.at[pl.ds(sid * EPS, EPS)],
                    hpart_hbm.at[pl.ds(cid * NPAD + sid * EPS, EPS)])
    pltpu.sync_copy(shared_e.at[pl.ds(sid * EPS, EPS)],
                    epart_hbm.at[pl.ds(cid * NPAD + sid * EPS, EPS)])


def _edge_pass(x, u, v, sd, zeros_h, zeros_pad):
    k = pl.kernel(
        _edge_kernel,
        out_type=(jax.ShapeDtypeStruct((NC * NPAD, D), jnp.float32),
                  jax.ShapeDtypeStruct((NC * NPAD,), jnp.float32)),
        mesh=_mesh(),
        compiler_params=pltpu.CompilerParams(needs_layout_passes=False),
        scratch_types=[
            pltpu.VMEM((2, 2, K), jnp.int32),
            pltpu.VMEM((2, K, D), jnp.float32),
            pltpu.VMEM((2, K), jnp.float32),
            pltpu.VMEM((2, 2, K), jnp.float32),
            pltpu.VMEM_SHARED((NPAD, D), jnp.float32),
            pltpu.VMEM_SHARED((NPAD,), jnp.float32),
        ] + [pltpu.SemaphoreType.DMA] * 10,
    )
    return k(x, u, v, sd, zeros_h, zeros_pad)


_GRID = 10
_RB = N // _GRID


def _prologue_kernel(degpair_ref, feats_ref, wc_ref, norm_ref, u_ref, v_ref):
    deg = degpair_ref[:, 0:1] + degpair_ref[:, 1:2]
    norm = lax.rsqrt(jnp.maximum(deg, 1.0))
    pq = jnp.dot(feats_ref[...], wc_ref[...], preferred_element_type=jnp.float32)
    norm_ref[...] = norm
    u_ref[...] = norm * pq[:, 0:1]
    v_ref[...] = norm * pq[:, 1:2]


def _prologue(degpair, feats, wcat):
    return pl.pallas_call(
        _prologue_kernel,
        grid=(_GRID,),
        in_specs=[
            pl.BlockSpec((_RB, 2), lambda i: (i, 0)),
            pl.BlockSpec((_RB, D), lambda i: (i, 0)),
            pl.BlockSpec((D, 2), lambda i: (0, 0)),
        ],
        out_specs=[
            pl.BlockSpec((_RB, 1), lambda i: (i, 0)),
            pl.BlockSpec((_RB, 1), lambda i: (i, 0)),
            pl.BlockSpec((_RB, 1), lambda i: (i, 0)),
        ],
        out_shape=[jax.ShapeDtypeStruct((N, 1), jnp.float32)] * 3,
    )(degpair, feats, wcat)


def _epilogue_kernel(hp_ref, ep_ref, y_ref, norm_ref, wc_ref, sc_ref,
                     x_ref, yo_ref, u_ref, v_ref):
    es = ep_ref[:, 0:1] + ep_ref[:, 1:2] + 1e-16
    h = (hp_ref[0] + hp_ref[1]) / es
    x_ref[...] = h
    yo_ref[...] = (y_ref[...] + h) * sc_ref[0, 0]
    norm = norm_ref[...]
    pq = jnp.dot(h, wc_ref[...], preferred_element_type=jnp.float32)
    u_ref[...] = norm * pq[:, 0:1]
    v_ref[...] = norm * pq[:, 1:2]


def _epilogue(hpart, epair, y_prev, norm, wcat, sc):
    return pl.pallas_call(
        _epilogue_kernel,
        grid=(_GRID,),
        in_specs=[
            pl.BlockSpec((NC, _RB, D), lambda i: (0, i, 0)),
            pl.BlockSpec((_RB, 2), lambda i: (i, 0)),
            pl.BlockSpec((_RB, D), lambda i: (i, 0)),
            pl.BlockSpec((_RB, 1), lambda i: (i, 0)),
            pl.BlockSpec((D, 2), lambda i: (0, 0)),
            pl.BlockSpec((1, 1), lambda i: (0, 0)),
        ],
        out_specs=[
            pl.BlockSpec((_RB, D), lambda i: (i, 0)),
            pl.BlockSpec((_RB, D), lambda i: (i, 0)),
            pl.BlockSpec((_RB, 1), lambda i: (i, 0)),
            pl.BlockSpec((_RB, 1), lambda i: (i, 0)),
        ],
        out_shape=[
            jax.ShapeDtypeStruct((N, D), jnp.float32),
            jax.ShapeDtypeStruct((N, D), jnp.float32),
            jax.ShapeDtypeStruct((N, 1), jnp.float32),
            jax.ShapeDtypeStruct((N, 1), jnp.float32),
        ],
    )(hpart, epair, y_prev, norm, wcat, sc)


def kernel(feats, edge_index, order, W_att):
    srcp = jnp.pad(edge_index[0].astype(jnp.int32).reshape(NW, EPT),
                   ((0, 0), (0, EPT2 - EPT)))
    dstp = jnp.pad(edge_index[1].astype(jnp.int32).reshape(NW, EPT),
                   ((0, 0), (0, EPT2 - EPT)), constant_values=N)
    sd = jnp.stack([srcp.reshape(NW, NCHUNK, K),
                    dstp.reshape(NW, NCHUNK, K)], axis=2)
    wcat = W_att[:, 0].reshape(2, D).T
    zeros_pad = jnp.zeros((NPAD,), jnp.float32)
    zeros_h = jnp.zeros((NPAD, D), jnp.float32)

    degpart = _deg_pass(sd, zeros_pad)
    degpair = degpart.reshape(NC, NPAD)[:, :N].T
    norm, u, v = _prologue(degpair, feats, wcat)

    one = jnp.ones((1, 1), jnp.float32)
    last = (1.0 / (order + 1.0)) * one

    x = feats
    y = feats
    for t in range(4):
        upad = jnp.concatenate([u.reshape(N), jnp.zeros(NPAD - N, jnp.float32)])
        vpad = jnp.concatenate([v.reshape(N), jnp.zeros(NPAD - N, jnp.float32)])
        hflat, eflat = _edge_pass(x, upad, vpad, sd, zeros_h, zeros_pad)
        hpart = hflat.reshape(NC, NPAD, D)
        epair = eflat.reshape(NC, NPAD)[:, :N].T
        sc = last if t == 3 else one
        x, y, u, v = _epilogue(hpart, epair, y, norm, wcat, sc)
    return y


# speedup vs baseline: 1.4143x; 1.0009x over previous
import jax
import jax.numpy as jnp
from jax import lax
from jax.experimental import pallas as pl
from jax.experimental.pallas import tpu as pltpu
from jax.experimental.pallas import tpu_sc as plsc

N = 10000
E = 320000
D = 128
NC = 2
NS = 16
NW = NC * NS
EPT = E // NW
K = 80
EPT2 = 10080
NCHUNK = EPT2 // K
NPAD = 10240
EPS = NPAD // NS


def _mesh():
    return plsc.VectorSubcoreMesh(core_axis_name="c", subcore_axis_name="s")


def _deg_kernel(sd_hbm, zero_hbm, degpart_hbm, sdl, ones_v, shared_deg):
    cid = lax.axis_index("c")
    sid = lax.axis_index("s")
    wid = sid * NC + cid
    pltpu.sync_copy(zero_hbm.at[pl.ds(sid * EPS, EPS)],
                    shared_deg.at[pl.ds(sid * EPS, EPS)])
    pltpu.sync_copy(sd_hbm.at[wid], sdl)
    for i in range(K // 16):
        ones_v[pl.ds(i * 16, 16)] = jnp.full((16,), 1.0, jnp.float32)
    plsc.subcore_barrier()

    def body(c, carry):
        pltpu.sync_copy(ones_v, shared_deg.at[sdl.at[c, 1]], add=True)
        return carry

    lax.fori_loop(0, NCHUNK, body, 0)
    plsc.subcore_barrier()
    pltpu.sync_copy(shared_deg.at[pl.ds(sid * EPS, EPS)],
                    degpart_hbm.at[pl.ds(cid * NPAD + sid * EPS, EPS)])


def _deg_pass(sd, zeros_pad):
    k = pl.kernel(
        _deg_kernel,
        out_type=jax.ShapeDtypeStruct((NC * NPAD,), jnp.float32),
        mesh=_mesh(),
        compiler_params=pltpu.CompilerParams(needs_layout_passes=False),
        scratch_types=[
            pltpu.VMEM((NCHUNK, 2, K), jnp.int32),
            pltpu.VMEM((K,), jnp.float32),
            pltpu.VMEM_SHARED((NPAD,), jnp.float32),
        ],
    )
    return k(sd, zeros_pad)


PAIRS = NCHUNK // 2


def _edge_kernel(x_hbm, u_hbm, v_hbm, sd_hbm, zh_hbm, ze_hbm,
                 hpart_hbm, epart_hbm,
                 sdbuf, rows2, ex2, scal, shared_h, shared_e,
                 sem_ra, sem_rb, sem_ua, sem_va, sem_ub, sem_vb,
                 sem_ea, sem_eb, sem_ha, sem_hb):
    cid = lax.axis_index("c")
    sid = lax.axis_index("s")
    wid = sid * NC + cid
    pltpu.sync_copy(zh_hbm.at[pl.ds(sid * EPS, EPS)],
                    shared_h.at[pl.ds(sid * EPS, EPS)])
    pltpu.sync_copy(ze_hbm.at[pl.ds(sid * EPS, EPS)],
                    shared_e.at[pl.ds(sid * EPS, EPS)])

    z16 = jnp.zeros((16,), jnp.float32)

    def zr(j, carry):
        for db in range(D // 16):
            rows2[0, j, pl.ds(db * 16, 16)] = z16
            rows2[1, j, pl.ds(db * 16, 16)] = z16
        return carry

    lax.fori_loop(0, K, zr, 0)
    for i in range(K // 16):
        ex2[0, pl.ds(i * 16, 16)] = z16
        ex2[1, pl.ds(i * 16, 16)] = z16
    plsc.subcore_barrier()

    pltpu.sync_copy(sd_hbm.at[wid, pl.ds(0, 2)], sdbuf)
    pltpu.async_copy(ex2.at[0], shared_e.at[sdbuf.at[0, 1]], sem_ea, add=True)
    pltpu.async_copy(ex2.at[1], shared_e.at[sdbuf.at[1, 1]], sem_eb, add=True)
    pltpu.async_copy(rows2.at[0], shared_h.at[sdbuf.at[0, 1]], sem_ha, add=True)
    pltpu.async_copy(rows2.at[1], shared_h.at[sdbuf.at[1, 1]], sem_hb, add=True)

    def process_slot(slot, du, dv, dr, se, sh):
        du.wait()
        dv.wait()
        for i in range(K // 16):
            e = scal[slot, 0, pl.ds(i * 16, 16)] + scal[slot, 1, pl.ds(i * 16, 16)]
            e = jnp.where(e >= 0.0, e, e * 0.2)
            ex2[slot, pl.ds(i * 16, 16)] = jnp.exp(e)
        pltpu.async_copy(ex2.at[slot], shared_e.at[sdbuf.at[slot, 1]], se,
                         add=True)
        dr.wait()

        def scale(kk, inner):
            s = plsc.load_gather(ex2.at[slot],
                                 [jnp.zeros((16,), jnp.int32) + kk])
            for db in range(D // 16):
                rows2[slot, kk, pl.ds(db * 16, 16)] = (
                    rows2[slot, kk, pl.ds(db * 16, 16)] * s)
            return inner

        lax.fori_loop(0, K, scale, 0)
        pltpu.async_copy(rows2.at[slot], shared_h.at[sdbuf.at[slot, 1]], sh,
                         add=True)

    def drain_all():
        pltpu.make_async_copy(ex2.at[0], shared_e.at[sdbuf.at[0, 1]], sem_ea).wait()
        pltpu.make_async_copy(ex2.at[1], shared_e.at[sdbuf.at[1, 1]], sem_eb).wait()
        pltpu.make_async_copy(rows2.at[0], shared_h.at[sdbuf.at[0, 1]], sem_ha).wait()
        pltpu.make_async_copy(rows2.at[1], shared_h.at[sdbuf.at[1, 1]], sem_hb).wait()

    def body(i, carry):
        drain_all()
        pltpu.sync_copy(sd_hbm.at[wid, pl.ds(2 * i, 2)], sdbuf)
        dra = pltpu.async_copy(x_hbm.at[sdbuf.at[0, 0]], rows2.at[0], sem_ra)
        drb = pltpu.async_copy(x_hbm.at[sdbuf.at[1, 0]], rows2.at[1], sem_rb)
        dua = pltpu.async_copy(u_hbm.at[sdbuf.at[0, 0]], scal.at[0, 0], sem_ua)
        dva = pltpu.async_copy(v_hbm.at[sdbuf.at[0, 1]], scal.at[0, 1], sem_va)
        dub = pltpu.async_copy(u_hbm.at[sdbuf.at[1, 0]], scal.at[1, 0], sem_ub)
        dvb = pltpu.async_copy(v_hbm.at[sdbuf.at[1, 1]], scal.at[1, 1], sem_vb)
        process_slot(0, dua, dva, dra, sem_ea, sem_ha)
        process_slot(1, dub, dvb, drb, sem_eb, sem_hb)
        return carry

    lax.fori_loop(0, PAIRS, body, 0)
    drain_all()
    plsc.subcore_barrier()
    pltpu.sync_copy(shared_h.at[pl.ds(sid * EPS, EPS)],
                    hpart_hbm.at[pl.ds(cid * NPAD + sid * EPS, EPS)])
    pltpu.sync_copy(shared_e.at[pl.ds(sid * EPS, EPS)],
                    epart_hbm.at[pl.ds(cid * NPAD + sid * EPS, EPS)])


def _edge_pass(x, u, v, sd, zeros_h, zeros_pad):
    k = pl.kernel(
        _edge_kernel,
        out_type=(jax.ShapeDtypeStruct((NC * NPAD, D), jnp.float32),
                  jax.ShapeDtypeStruct((NC * NPAD,), jnp.float32)),
        mesh=_mesh(),
        compiler_params=pltpu.CompilerParams(needs_layout_passes=False),
        scratch_types=[
            pltpu.VMEM((2, 2, K), jnp.int32),
            pltpu.VMEM((2, K, D), jnp.float32),
            pltpu.VMEM((2, K), jnp.float32),
            pltpu.VMEM((2, 2, K), jnp.float32),
            pltpu.VMEM_SHARED((NPAD, D), jnp.float32),
            pltpu.VMEM_SHARED((NPAD,), jnp.float32),
        ] + [pltpu.SemaphoreType.DMA] * 10,
    )
    return k(x, u, v, sd, zeros_h, zeros_pad)


_GRID = 10
_RB = N // _GRID


def _prologue_kernel(degpair_ref, feats_ref, wc_ref, norm_ref, u_ref, v_ref):
    deg = degpair_ref[:, 0:1] + degpair_ref[:, 1:2]
    norm = lax.rsqrt(jnp.maximum(deg, 1.0))
    pq = jnp.dot(feats_ref[...], wc_ref[...], preferred_element_type=jnp.float32)
    norm_ref[...] = norm
    u_ref[...] = norm * pq[:, 0:1]
    v_ref[...] = norm * pq[:, 1:2]


def _prologue(degpair, feats, wcat):
    return pl.pallas_call(
        _prologue_kernel,
        grid=(_GRID,),
        in_specs=[
            pl.BlockSpec((_RB, 2), lambda i: (i, 0)),
            pl.BlockSpec((_RB, D), lambda i: (i, 0)),
            pl.BlockSpec((D, 2), lambda i: (0, 0)),
        ],
        out_specs=[
            pl.BlockSpec((_RB, 1), lambda i: (i, 0)),
            pl.BlockSpec((_RB, 1), lambda i: (i, 0)),
            pl.BlockSpec((_RB, 1), lambda i: (i, 0)),
        ],
        out_shape=[jax.ShapeDtypeStruct((N, 1), jnp.float32)] * 3,
    )(degpair, feats, wcat)


def _epilogue_kernel(hp_ref, ep_ref, y_ref, norm_ref, wc_ref, sc_ref,
                     x_ref, yo_ref, u_ref, v_ref):
    es = ep_ref[:, 0:1] + ep_ref[:, 1:2] + 1e-16
    h = (hp_ref[0] + hp_ref[1]) / es
    x_ref[...] = h
    yo_ref[...] = (y_ref[...] + h) * sc_ref[0, 0]
    norm = norm_ref[...]
    pq = jnp.dot(h, wc_ref[...], preferred_element_type=jnp.float32)
    u_ref[...] = norm * pq[:, 0:1]
    v_ref[...] = norm * pq[:, 1:2]


def _epilogue(hpart, epair, y_prev, norm, wcat, sc):
    return pl.pallas_call(
        _epilogue_kernel,
        grid=(_GRID,),
        in_specs=[
            pl.BlockSpec((NC, _RB, D), lambda i: (0, i, 0)),
            pl.BlockSpec((_RB, 2), lambda i: (i, 0)),
            pl.BlockSpec((_RB, D), lambda i: (i, 0)),
            pl.BlockSpec((_RB, 1), lambda i: (i, 0)),
            pl.BlockSpec((D, 2), lambda i: (0, 0)),
            pl.BlockSpec((1, 1), lambda i: (0, 0)),
        ],
        out_specs=[
            pl.BlockSpec((_RB, D), lambda i: (i, 0)),
            pl.BlockSpec((_RB, D), lambda i: (i, 0)),
            pl.BlockSpec((_RB, 1), lambda i: (i, 0)),
            pl.BlockSpec((_RB, 1), lambda i: (i, 0)),
        ],
        out_shape=[
            jax.ShapeDtypeStruct((N, D), jnp.float32),
            jax.ShapeDtypeStruct((N, D), jnp.float32),
            jax.ShapeDtypeStruct((N, 1), jnp.float32),
            jax.ShapeDtypeStruct((N, 1), jnp.float32),
        ],
    )(hpart, epair, y_prev, norm, wcat, sc)


def kernel(feats, edge_index, order, W_att):
    srcp = jnp.pad(edge_index[0].astype(jnp.int32).reshape(NW, EPT),
                   ((0, 0), (0, EPT2 - EPT)))
    fake_dst = jnp.broadcast_to(N + jnp.arange(NW, dtype=jnp.int32)[:, None],
                                (NW, EPT2 - EPT))
    dstp = jnp.concatenate(
        [edge_index[1].astype(jnp.int32).reshape(NW, EPT), fake_dst], axis=1)
    sd = jnp.stack([srcp.reshape(NW, NCHUNK, K),
                    dstp.reshape(NW, NCHUNK, K)], axis=2)
    wcat = W_att[:, 0].reshape(2, D).T
    zeros_pad = jnp.zeros((NPAD,), jnp.float32)
    zeros_h = jnp.zeros((NPAD, D), jnp.float32)

    degpart = _deg_pass(sd, zeros_pad)
    degpair = degpart.reshape(NC, NPAD)[:, :N].T
    norm, u, v = _prologue(degpair, feats, wcat)

    one = jnp.ones((1, 1), jnp.float32)
    last = (1.0 / (order + 1.0)) * one

    x = feats
    y = feats
    for t in range(4):
        upad = jnp.concatenate([u.reshape(N), jnp.zeros(NPAD - N, jnp.float32)])
        vpad = jnp.concatenate([v.reshape(N), jnp.zeros(NPAD - N, jnp.float32)])
        hflat, eflat = _edge_pass(x, upad, vpad, sd, zeros_h, zeros_pad)
        hpart = hflat.reshape(NC, NPAD, D)
        epair = eflat.reshape(NC, NPAD)[:, :N].T
        sc = last if t == 3 else one
        x, y, u, v = _epilogue(hpart, epair, y, norm, wcat, sc)
    return y


# speedup vs baseline: 1.7831x; 1.2608x over previous
import jax
import jax.numpy as jnp
from jax import lax
from jax.experimental import pallas as pl
from jax.experimental.pallas import tpu as pltpu
from jax.experimental.pallas import tpu_sc as plsc

N = 10000
E = 320000
D = 128
NC = 2
NS = 16
NW = NC * NS
EPT = E // NW
K = 80
EPT2 = 10080
NCHUNK = EPT2 // K
NPAD = 10240
EPS = NPAD // NS


def _mesh():
    return plsc.VectorSubcoreMesh(core_axis_name="c", subcore_axis_name="s")


def _deg_kernel(sd_hbm, zero_hbm, degpart_hbm, sdl, ones_v, shared_deg):
    cid = lax.axis_index("c")
    sid = lax.axis_index("s")
    wid = sid * NC + cid
    pltpu.sync_copy(zero_hbm.at[pl.ds(sid * EPS, EPS)],
                    shared_deg.at[pl.ds(sid * EPS, EPS)])
    pltpu.sync_copy(sd_hbm.at[wid], sdl)
    for i in range(K // 16):
        ones_v[pl.ds(i * 16, 16)] = jnp.full((16,), 1.0, jnp.float32)
    plsc.subcore_barrier()

    def body(c, carry):
        pltpu.sync_copy(ones_v, shared_deg.at[sdl.at[c, 1]], add=True)
        return carry

    lax.fori_loop(0, NCHUNK, body, 0)
    plsc.subcore_barrier()
    pltpu.sync_copy(shared_deg.at[pl.ds(sid * EPS, EPS)],
                    degpart_hbm.at[pl.ds(cid * NPAD + sid * EPS, EPS)])


def _deg_pass(sd, zeros_pad):
    k = pl.kernel(
        _deg_kernel,
        out_type=jax.ShapeDtypeStruct((NC * NPAD,), jnp.float32),
        mesh=_mesh(),
        compiler_params=pltpu.CompilerParams(needs_layout_passes=False),
        scratch_types=[
            pltpu.VMEM((NCHUNK, 2, K), jnp.int32),
            pltpu.VMEM((K,), jnp.float32),
            pltpu.VMEM_SHARED((NPAD,), jnp.float32),
        ],
    )
    return k(sd, zeros_pad)


PAIRS = NCHUNK // 2


def _edge_kernel(x_hbm, u_hbm, v_hbm, sd_hbm, zh_hbm, ze_hbm,
                 hpart_hbm, epart_hbm,
                 sdbuf, rows2, ex2, scal, shared_h, shared_e,
                 sem_ra, sem_rb, sem_ua, sem_va, sem_ub, sem_vb,
                 sem_ea, sem_eb, sem_ha, sem_hb):
    cid = lax.axis_index("c")
    sid = lax.axis_index("s")
    wid = sid * NC + cid
    pltpu.sync_copy(zh_hbm.at[pl.ds(sid * EPS, EPS)],
                    shared_h.at[pl.ds(sid * EPS, EPS)])
    pltpu.sync_copy(ze_hbm.at[pl.ds(sid * EPS, EPS)],
                    shared_e.at[pl.ds(sid * EPS, EPS)])

    z16 = jnp.zeros((16,), jnp.float32)

    def zr(j, carry):
        for db in range(D // 16):
            rows2[0, j, pl.ds(db * 16, 16)] = z16
            rows2[1, j, pl.ds(db * 16, 16)] = z16
        return carry

    lax.fori_loop(0, K, zr, 0)
    for i in range(K // 16):
        ex2[0, pl.ds(i * 16, 16)] = z16
        ex2[1, pl.ds(i * 16, 16)] = z16
    plsc.subcore_barrier()

    pltpu.sync_copy(sd_hbm.at[wid, pl.ds(0, 2)], sdbuf)
    pltpu.async_copy(ex2.at[0], shared_e.at[sdbuf.at[0, 1]], sem_ea, add=True)
    pltpu.async_copy(ex2.at[1], shared_e.at[sdbuf.at[1, 1]], sem_eb, add=True)
    pltpu.async_copy(rows2.at[0], shared_h.at[sdbuf.at[0, 1]], sem_ha, add=True)
    pltpu.async_copy(rows2.at[1], shared_h.at[sdbuf.at[1, 1]], sem_hb, add=True)

    def process_slot(slot, du, dv, dr, se, sh):
        du.wait()
        dv.wait()
        for i in range(K // 16):
            e = scal[slot, 0, pl.ds(i * 16, 16)] + scal[slot, 1, pl.ds(i * 16, 16)]
            e = jnp.where(e >= 0.0, e, e * 0.2)
            ex2[slot, pl.ds(i * 16, 16)] = jnp.exp(e)
        pltpu.async_copy(ex2.at[slot], shared_e.at[sdbuf.at[slot, 1]], se,
                         add=True)
        dr.wait()

        def scale(kk, inner):
            s = plsc.load_gather(ex2.at[slot],
                                 [jnp.zeros((16,), jnp.int32) + kk])
            for db in range(D // 16):
                rows2[slot, kk, pl.ds(db * 16, 16)] = (
                    rows2[slot, kk, pl.ds(db * 16, 16)] * s)
            return inner

        lax.fori_loop(0, K, scale, 0)
        pltpu.async_copy(rows2.at[slot], shared_h.at[sdbuf.at[slot, 1]], sh,
                         add=True)

    def drain_all():
        pltpu.make_async_copy(ex2.at[0], shared_e.at[sdbuf.at[0, 1]], sem_ea).wait()
        pltpu.make_async_copy(ex2.at[1], shared_e.at[sdbuf.at[1, 1]], sem_eb).wait()
        pltpu.make_async_copy(rows2.at[0], shared_h.at[sdbuf.at[0, 1]], sem_ha).wait()
        pltpu.make_async_copy(rows2.at[1], shared_h.at[sdbuf.at[1, 1]], sem_hb).wait()

    def body(i, carry):
        drain_all()
        pltpu.sync_copy(sd_hbm.at[wid, pl.ds(2 * i, 2)], sdbuf)
        dra = pltpu.async_copy(x_hbm.at[sdbuf.at[0, 0]], rows2.at[0], sem_ra)
        drb = pltpu.async_copy(x_hbm.at[sdbuf.at[1, 0]], rows2.at[1], sem_rb)
        dua = pltpu.async_copy(u_hbm.at[sdbuf.at[0, 0]], scal.at[0, 0], sem_ua)
        dva = pltpu.async_copy(v_hbm.at[sdbuf.at[0, 1]], scal.at[0, 1], sem_va)
        dub = pltpu.async_copy(u_hbm.at[sdbuf.at[1, 0]], scal.at[1, 0], sem_ub)
        dvb = pltpu.async_copy(v_hbm.at[sdbuf.at[1, 1]], scal.at[1, 1], sem_vb)
        process_slot(0, dua, dva, dra, sem_ea, sem_ha)
        process_slot(1, dub, dvb, drb, sem_eb, sem_hb)
        return carry

    lax.fori_loop(0, PAIRS, body, 0)
    drain_all()
    plsc.subcore_barrier()
    pltpu.sync_copy(shared_h.at[pl.ds(sid * EPS, EPS)],
                    hpart_hbm.at[pl.ds(cid * NPAD + sid * EPS, EPS)])
    pltpu.sync_copy(shared_e.at[pl.ds(sid * EPS, EPS)],
                    epart_hbm.at[pl.ds(cid * NPAD + sid * EPS, EPS)])


def _edge_pass(x, u, v, sd, zeros_h, zeros_pad):
    k = pl.kernel(
        _edge_kernel,
        out_type=(jax.ShapeDtypeStruct((NC * NPAD, D), jnp.float32),
                  jax.ShapeDtypeStruct((NC * NPAD,), jnp.float32)),
        mesh=_mesh(),
        compiler_params=pltpu.CompilerParams(needs_layout_passes=False),
        scratch_types=[
            pltpu.VMEM((2, 2, K), jnp.int32),
            pltpu.VMEM((2, K, D), jnp.float32),
            pltpu.VMEM((2, K), jnp.float32),
            pltpu.VMEM((2, 2, K), jnp.float32),
            pltpu.VMEM_SHARED((NPAD, D), jnp.float32),
            pltpu.VMEM_SHARED((NPAD,), jnp.float32),
        ] + [pltpu.SemaphoreType.DMA] * 10,
    )
    return k(x, u, v, sd, zeros_h, zeros_pad)


_GRID = 10
_RB = N // _GRID


def _prologue_kernel(degpair_ref, feats_ref, wc_ref, norm_ref, u_ref, v_ref):
    deg = degpair_ref[:, 0:1] + degpair_ref[:, 1:2]
    norm = lax.rsqrt(jnp.maximum(deg, 1.0))
    pq = jnp.dot(feats_ref[...], wc_ref[...], preferred_element_type=jnp.float32)
    norm_ref[...] = norm
    u_ref[...] = norm * pq[:, 0:1]
    v_ref[...] = norm * pq[:, 1:2]


def _prologue(degpair, feats, wcat):
    return pl.pallas_call(
        _prologue_kernel,
        grid=(_GRID,),
        in_specs=[
            pl.BlockSpec((_RB, 2), lambda i: (i, 0)),
            pl.BlockSpec((_RB, D), lambda i: (i, 0)),
            pl.BlockSpec((D, 2), lambda i: (0, 0)),
        ],
        out_specs=[
            pl.BlockSpec((_RB, 1), lambda i: (i, 0)),
            pl.BlockSpec((_RB, 1), lambda i: (i, 0)),
            pl.BlockSpec((_RB, 1), lambda i: (i, 0)),
        ],
        out_shape=[jax.ShapeDtypeStruct((N, 1), jnp.float32)] * 3,
    )(degpair, feats, wcat)


def _epilogue_kernel(hp_ref, ep_ref, y_ref, norm_ref, wc_ref, sc_ref,
                     x_ref, yo_ref, u_ref, v_ref):
    es = ep_ref[:, 0:1] + ep_ref[:, 1:2] + 1e-16
    h = (hp_ref[0] + hp_ref[1]) / es
    x_ref[...] = h
    yo_ref[...] = (y_ref[...] + h) * sc_ref[0, 0]
    norm = norm_ref[...]
    pq = jnp.dot(h, wc_ref[...], preferred_element_type=jnp.float32)
    u_ref[...] = norm * pq[:, 0:1]
    v_ref[...] = norm * pq[:, 1:2]


def _epilogue(hpart, epair, y_prev, norm, wcat, sc):
    return pl.pallas_call(
        _epilogue_kernel,
        grid=(_GRID,),
        in_specs=[
            pl.BlockSpec((NC, _RB, D), lambda i: (0, i, 0)),
            pl.BlockSpec((_RB, 2), lambda i: (i, 0)),
            pl.BlockSpec((_RB, D), lambda i: (i, 0)),
            pl.BlockSpec((_RB, 1), lambda i: (i, 0)),
            pl.BlockSpec((D, 2), lambda i: (0, 0)),
            pl.BlockSpec((1, 1), lambda i: (0, 0)),
        ],
        out_specs=[
            pl.BlockSpec((_RB, D), lambda i: (i, 0)),
            pl.BlockSpec((_RB, D), lambda i: (i, 0)),
            pl.BlockSpec((_RB, 1), lambda i: (i, 0)),
            pl.BlockSpec((_RB, 1), lambda i: (i, 0)),
        ],
        out_shape=[
            jax.ShapeDtypeStruct((N, D), jnp.float32),
            jax.ShapeDtypeStruct((N, D), jnp.float32),
            jax.ShapeDtypeStruct((N, 1), jnp.float32),
            jax.ShapeDtypeStruct((N, 1), jnp.float32),
        ],
    )(hpart, epair, y_prev, norm, wcat, sc)


def kernel(feats, edge_index, order, W_att):
    pad_n = EPT2 - EPT
    j = jnp.arange(pad_n, dtype=jnp.int32)[None, :]
    w = jnp.arange(NW, dtype=jnp.int32)[:, None]
    fake_src = (w * 311 + j * 37) % N
    fake_dst = N + (w * pad_n + j) % (NPAD - N)
    srcp = jnp.concatenate(
        [edge_index[0].astype(jnp.int32).reshape(NW, EPT),
         jnp.broadcast_to(fake_src, (NW, pad_n))], axis=1)
    dstp = jnp.concatenate(
        [edge_index[1].astype(jnp.int32).reshape(NW, EPT),
         jnp.broadcast_to(fake_dst, (NW, pad_n))], axis=1)
    sd = jnp.stack([srcp.reshape(NW, NCHUNK, K),
                    dstp.reshape(NW, NCHUNK, K)], axis=2)
    wcat = W_att[:, 0].reshape(2, D).T
    zeros_pad = jnp.zeros((NPAD,), jnp.float32)
    zeros_h = jnp.zeros((NPAD, D), jnp.float32)

    degpart = _deg_pass(sd, zeros_pad)
    degpair = degpart.reshape(NC, NPAD)[:, :N].T
    norm, u, v = _prologue(degpair, feats, wcat)

    one = jnp.ones((1, 1), jnp.float32)
    last = (1.0 / (order + 1.0)) * one

    x = feats
    y = feats
    for t in range(4):
        upad = jnp.concatenate([u.reshape(N), jnp.zeros(NPAD - N, jnp.float32)])
        vpad = jnp.concatenate([v.reshape(N), jnp.zeros(NPAD - N, jnp.float32)])
        hflat, eflat = _edge_pass(x, upad, vpad, sd, zeros_h, zeros_pad)
        hpart = hflat.reshape(NC, NPAD, D)
        epair = eflat.reshape(NC, NPAD)[:, :N].T
        sc = last if t == 3 else one
        x, y, u, v = _epilogue(hpart, epair, y, norm, wcat, sc)
    return y


# speedup vs baseline: 1.8578x; 1.0419x over previous
import jax
import jax.numpy as jnp
from jax import lax
from jax.experimental import pallas as pl
from jax.experimental.pallas import tpu as pltpu
from jax.experimental.pallas import tpu_sc as plsc

N = 10000
E = 320000
D = 128
NC = 2
NS = 16
NW = NC * NS
EPT = E // NW
K = 128
EPT2 = 10240
NCHUNK = EPT2 // K
NPAD = 10240
EPS = NPAD // NS


def _mesh():
    return plsc.VectorSubcoreMesh(core_axis_name="c", subcore_axis_name="s")


def _deg_kernel(sd_hbm, zero_hbm, degpart_hbm, sdl, ones_v, shared_deg):
    cid = lax.axis_index("c")
    sid = lax.axis_index("s")
    wid = sid * NC + cid
    pltpu.sync_copy(zero_hbm.at[pl.ds(sid * EPS, EPS)],
                    shared_deg.at[pl.ds(sid * EPS, EPS)])
    pltpu.sync_copy(sd_hbm.at[wid], sdl)
    for i in range(K // 16):
        ones_v[pl.ds(i * 16, 16)] = jnp.full((16,), 1.0, jnp.float32)
    plsc.subcore_barrier()

    def body(c, carry):
        pltpu.sync_copy(ones_v, shared_deg.at[sdl.at[c, 1]], add=True)
        return carry

    lax.fori_loop(0, NCHUNK, body, 0)
    plsc.subcore_barrier()
    pltpu.sync_copy(shared_deg.at[pl.ds(sid * EPS, EPS)],
                    degpart_hbm.at[pl.ds(cid * NPAD + sid * EPS, EPS)])


def _deg_pass(sd, zeros_pad):
    k = pl.kernel(
        _deg_kernel,
        out_type=jax.ShapeDtypeStruct((NC * NPAD,), jnp.float32),
        mesh=_mesh(),
        compiler_params=pltpu.CompilerParams(needs_layout_passes=False),
        scratch_types=[
            pltpu.VMEM((NCHUNK, 2, K), jnp.int32),
            pltpu.VMEM((K,), jnp.float32),
            pltpu.VMEM_SHARED((NPAD,), jnp.float32),
        ],
    )
    return k(sd, zeros_pad)


PAIRS = NCHUNK // 2


def _edge_kernel(x_hbm, u_hbm, v_hbm, sd_hbm, zh_hbm, ze_hbm,
                 hpart_hbm, epart_hbm,
                 sdbuf, rows2, ex2, scal, shared_h, shared_e,
                 sem_ra, sem_rb, sem_ua, sem_va, sem_ub, sem_vb,
                 sem_ea, sem_eb, sem_ha, sem_hb):
    cid = lax.axis_index("c")
    sid = lax.axis_index("s")
    wid = sid * NC + cid
    pltpu.sync_copy(zh_hbm.at[pl.ds(sid * EPS, EPS)],
                    shared_h.at[pl.ds(sid * EPS, EPS)])
    pltpu.sync_copy(ze_hbm.at[pl.ds(sid * EPS, EPS)],
                    shared_e.at[pl.ds(sid * EPS, EPS)])

    z16 = jnp.zeros((16,), jnp.float32)

    def zr(j, carry):
        for db in range(D // 16):
            rows2[0, j, pl.ds(db * 16, 16)] = z16
            rows2[1, j, pl.ds(db * 16, 16)] = z16
        return carry

    lax.fori_loop(0, K, zr, 0)
    for i in range(K // 16):
        ex2[0, pl.ds(i * 16, 16)] = z16
        ex2[1, pl.ds(i * 16, 16)] = z16
    plsc.subcore_barrier()

    pltpu.sync_copy(sd_hbm.at[wid, pl.ds(0, 2)], sdbuf)
    pltpu.async_copy(ex2.at[0], shared_e.at[sdbuf.at[0, 1]], sem_ea, add=True)
    pltpu.async_copy(ex2.at[1], shared_e.at[sdbuf.at[1, 1]], sem_eb, add=True)
    pltpu.async_copy(rows2.at[0], shared_h.at[sdbuf.at[0, 1]], sem_ha, add=True)
    pltpu.async_copy(rows2.at[1], shared_h.at[sdbuf.at[1, 1]], sem_hb, add=True)

    def process_slot(slot, du, dv, dr, se, sh):
        du.wait()
        dv.wait()
        for i in range(K // 16):
            e = scal[slot, 0, pl.ds(i * 16, 16)] + scal[slot, 1, pl.ds(i * 16, 16)]
            e = jnp.where(e >= 0.0, e, e * 0.2)
            ex2[slot, pl.ds(i * 16, 16)] = jnp.exp(e)
        pltpu.async_copy(ex2.at[slot], shared_e.at[sdbuf.at[slot, 1]], se,
                         add=True)
        dr.wait()

        def scale(kk, inner):
            s = plsc.load_gather(ex2.at[slot],
                                 [jnp.zeros((16,), jnp.int32) + kk])
            for db in range(D // 16):
                rows2[slot, kk, pl.ds(db * 16, 16)] = (
                    rows2[slot, kk, pl.ds(db * 16, 16)] * s)
            return inner

        lax.fori_loop(0, K, scale, 0)
        pltpu.async_copy(rows2.at[slot], shared_h.at[sdbuf.at[slot, 1]], sh,
                         add=True)

    def drain_all():
        pltpu.make_async_copy(ex2.at[0], shared_e.at[sdbuf.at[0, 1]], sem_ea).wait()
        pltpu.make_async_copy(ex2.at[1], shared_e.at[sdbuf.at[1, 1]], sem_eb).wait()
        pltpu.make_async_copy(rows2.at[0], shared_h.at[sdbuf.at[0, 1]], sem_ha).wait()
        pltpu.make_async_copy(rows2.at[1], shared_h.at[sdbuf.at[1, 1]], sem_hb).wait()

    def body(i, carry):
        drain_all()
        pltpu.sync_copy(sd_hbm.at[wid, pl.ds(2 * i, 2)], sdbuf)
        dra = pltpu.async_copy(x_hbm.at[sdbuf.at[0, 0]], rows2.at[0], sem_ra)
        drb = pltpu.async_copy(x_hbm.at[sdbuf.at[1, 0]], rows2.at[1], sem_rb)
        dua = pltpu.async_copy(u_hbm.at[sdbuf.at[0, 0]], scal.at[0, 0], sem_ua)
        dva = pltpu.async_copy(v_hbm.at[sdbuf.at[0, 1]], scal.at[0, 1], sem_va)
        dub = pltpu.async_copy(u_hbm.at[sdbuf.at[1, 0]], scal.at[1, 0], sem_ub)
        dvb = pltpu.async_copy(v_hbm.at[sdbuf.at[1, 1]], scal.at[1, 1], sem_vb)
        process_slot(0, dua, dva, dra, sem_ea, sem_ha)
        process_slot(1, dub, dvb, drb, sem_eb, sem_hb)
        return carry

    lax.fori_loop(0, PAIRS, body, 0)
    drain_all()
    plsc.subcore_barrier()
    pltpu.sync_copy(shared_h.at[pl.ds(sid * EPS, EPS)],
                    hpart_hbm.at[pl.ds(cid * NPAD + sid * EPS, EPS)])
    pltpu.sync_copy(shared_e.at[pl.ds(sid * EPS, EPS)],
                    epart_hbm.at[pl.ds(cid * NPAD + sid * EPS, EPS)])


def _edge_pass(x, u, v, sd, zeros_h, zeros_pad):
    k = pl.kernel(
        _edge_kernel,
        out_type=(jax.ShapeDtypeStruct((NC * NPAD, D), jnp.float32),
                  jax.ShapeDtypeStruct((NC * NPAD,), jnp.float32)),
        mesh=_mesh(),
        compiler_params=pltpu.CompilerParams(needs_layout_passes=False),
        scratch_types=[
            pltpu.VMEM((2, 2, K), jnp.int32),
            pltpu.VMEM((2, K, D), jnp.float32),
            pltpu.VMEM((2, K), jnp.float32),
            pltpu.VMEM((2, 2, K), jnp.float32),
            pltpu.VMEM_SHARED((NPAD, D), jnp.float32),
            pltpu.VMEM_SHARED((NPAD,), jnp.float32),
        ] + [pltpu.SemaphoreType.DMA] * 10,
    )
    return k(x, u, v, sd, zeros_h, zeros_pad)


_GRID = 10
_RB = N // _GRID


def _prologue_kernel(degpair_ref, feats_ref, wc_ref, norm_ref, u_ref, v_ref):
    deg = degpair_ref[:, 0:1] + degpair_ref[:, 1:2]
    norm = lax.rsqrt(jnp.maximum(deg, 1.0))
    pq = jnp.dot(feats_ref[...], wc_ref[...], preferred_element_type=jnp.float32)
    norm_ref[...] = norm
    u_ref[...] = norm * pq[:, 0:1]
    v_ref[...] = norm * pq[:, 1:2]


def _prologue(degpair, feats, wcat):
    return pl.pallas_call(
        _prologue_kernel,
        grid=(_GRID,),
        in_specs=[
            pl.BlockSpec((_RB, 2), lambda i: (i, 0)),
            pl.BlockSpec((_RB, D), lambda i: (i, 0)),
            pl.BlockSpec((D, 2), lambda i: (0, 0)),
        ],
        out_specs=[
            pl.BlockSpec((_RB, 1), lambda i: (i, 0)),
            pl.BlockSpec((_RB, 1), lambda i: (i, 0)),
            pl.BlockSpec((_RB, 1), lambda i: (i, 0)),
        ],
        out_shape=[jax.ShapeDtypeStruct((N, 1), jnp.float32)] * 3,
    )(degpair, feats, wcat)


def _epilogue_kernel(hp_ref, ep_ref, y_ref, norm_ref, wc_ref, sc_ref,
                     x_ref, yo_ref, u_ref, v_ref):
    es = ep_ref[:, 0:1] + ep_ref[:, 1:2] + 1e-16
    h = (hp_ref[0] + hp_ref[1]) / es
    x_ref[...] = h
    yo_ref[...] = (y_ref[...] + h) * sc_ref[0, 0]
    norm = norm_ref[...]
    pq = jnp.dot(h, wc_ref[...], preferred_element_type=jnp.float32)
    u_ref[...] = norm * pq[:, 0:1]
    v_ref[...] = norm * pq[:, 1:2]


def _epilogue(hpart, epair, y_prev, norm, wcat, sc):
    return pl.pallas_call(
        _epilogue_kernel,
        grid=(_GRID,),
        in_specs=[
            pl.BlockSpec((NC, _RB, D), lambda i: (0, i, 0)),
            pl.BlockSpec((_RB, 2), lambda i: (i, 0)),
            pl.BlockSpec((_RB, D), lambda i: (i, 0)),
            pl.BlockSpec((_RB, 1), lambda i: (i, 0)),
            pl.BlockSpec((D, 2), lambda i: (0, 0)),
            pl.BlockSpec((1, 1), lambda i: (0, 0)),
        ],
        out_specs=[
            pl.BlockSpec((_RB, D), lambda i: (i, 0)),
            pl.BlockSpec((_RB, D), lambda i: (i, 0)),
            pl.BlockSpec((_RB, 1), lambda i: (i, 0)),
            pl.BlockSpec((_RB, 1), lambda i: (i, 0)),
        ],
        out_shape=[
            jax.ShapeDtypeStruct((N, D), jnp.float32),
            jax.ShapeDtypeStruct((N, D), jnp.float32),
            jax.ShapeDtypeStruct((N, 1), jnp.float32),
            jax.ShapeDtypeStruct((N, 1), jnp.float32),
        ],
    )(hpart, epair, y_prev, norm, wcat, sc)


def kernel(feats, edge_index, order, W_att):
    pad_n = EPT2 - EPT
    j = jnp.arange(pad_n, dtype=jnp.int32)[None, :]
    w = jnp.arange(NW, dtype=jnp.int32)[:, None]
    fake_src = (w * 311 + j * 37) % N
    fake_dst = N + (w * pad_n + j) % (NPAD - N)
    srcp = jnp.concatenate(
        [edge_index[0].astype(jnp.int32).reshape(NW, EPT),
         jnp.broadcast_to(fake_src, (NW, pad_n))], axis=1)
    dstp = jnp.concatenate(
        [edge_index[1].astype(jnp.int32).reshape(NW, EPT),
         jnp.broadcast_to(fake_dst, (NW, pad_n))], axis=1)
    sd = jnp.stack([srcp.reshape(NW, NCHUNK, K),
                    dstp.reshape(NW, NCHUNK, K)], axis=2)
    wcat = W_att[:, 0].reshape(2, D).T
    zeros_pad = jnp.zeros((NPAD,), jnp.float32)
    zeros_h = jnp.zeros((NPAD, D), jnp.float32)

    degpart = _deg_pass(sd, zeros_pad)
    degpair = degpart.reshape(NC, NPAD)[:, :N].T
    norm, u, v = _prologue(degpair, feats, wcat)

    one = jnp.ones((1, 1), jnp.float32)
    last = (1.0 / (order + 1.0)) * one

    x = feats
    y = feats
    for t in range(4):
        upad = jnp.concatenate([u.reshape(N), jnp.zeros(NPAD - N, jnp.float32)])
        vpad = jnp.concatenate([v.reshape(N), jnp.zeros(NPAD - N, jnp.float32)])
        hflat, eflat = _edge_pass(x, upad, vpad, sd, zeros_h, zeros_pad)
        hpart = hflat.reshape(NC, NPAD, D)
        epair = eflat.reshape(NC, NPAD)[:, :N].T
        sc = last if t == 3 else one
        x, y, u, v = _epilogue(hpart, epair, y, norm, wcat, sc)
    return y


# speedup vs baseline: 2.4460x; 1.3166x over previous
import jax
import jax.numpy as jnp
from jax import lax
from jax.experimental import pallas as pl
from jax.experimental.pallas import tpu as pltpu
from jax.experimental.pallas import tpu_sc as plsc

N = 10000
E = 320000
D = 128
NC = 2
NS = 16
NW = NC * NS
EPT = E // NW
K = 80
EPT2 = 10080
NCHUNK = EPT2 // K
NPAD = 10240
EPS = NPAD // NS


def _mesh():
    return plsc.VectorSubcoreMesh(core_axis_name="c", subcore_axis_name="s")


def _deg_kernel(sd_hbm, zero_hbm, degpart_hbm, sdl, ones_v, shared_deg):
    cid = lax.axis_index("c")
    sid = lax.axis_index("s")
    wid = sid * NC + cid
    pltpu.sync_copy(zero_hbm.at[pl.ds(sid * EPS, EPS)],
                    shared_deg.at[pl.ds(sid * EPS, EPS)])
    pltpu.sync_copy(sd_hbm.at[wid], sdl)
    for i in range(K // 16):
        ones_v[pl.ds(i * 16, 16)] = jnp.full((16,), 1.0, jnp.float32)
    plsc.subcore_barrier()

    def body(c, carry):
        pltpu.sync_copy(ones_v, shared_deg.at[sdl.at[c, 1]], add=True)
        return carry

    lax.fori_loop(0, NCHUNK, body, 0)
    plsc.subcore_barrier()
    pltpu.sync_copy(shared_deg.at[pl.ds(sid * EPS, EPS)],
                    degpart_hbm.at[pl.ds(cid * NPAD + sid * EPS, EPS)])


def _deg_pass(sd, zeros_pad):
    k = pl.kernel(
        _deg_kernel,
        out_type=jax.ShapeDtypeStruct((NC * NPAD,), jnp.float32),
        mesh=_mesh(),
        compiler_params=pltpu.CompilerParams(needs_layout_passes=False),
        scratch_types=[
            pltpu.VMEM((NCHUNK, 2, K), jnp.int32),
            pltpu.VMEM((K,), jnp.float32),
            pltpu.VMEM_SHARED((NPAD,), jnp.float32),
        ],
    )
    return k(sd, zeros_pad)


NBODY = NCHUNK // 3


def _edge_kernel(x_hbm, u_hbm, v_hbm, sd_hbm, zh_hbm, ze_hbm,
                 hpart_hbm, epart_hbm,
                 sd3, rows3, ex3, scal3, shared_h, shared_e, *sems):
    sem_r = sems[0:3]
    sem_u = sems[3:6]
    sem_v = sems[6:9]
    sem_e = sems[9:12]
    sem_h = sems[12:15]
    cid = lax.axis_index("c")
    sid = lax.axis_index("s")
    wid = sid * NC + cid
    pltpu.sync_copy(zh_hbm.at[pl.ds(sid * EPS, EPS)],
                    shared_h.at[pl.ds(sid * EPS, EPS)])
    pltpu.sync_copy(ze_hbm.at[pl.ds(sid * EPS, EPS)],
                    shared_e.at[pl.ds(sid * EPS, EPS)])

    z16 = jnp.zeros((16,), jnp.float32)

    def zr(j, carry):
        for db in range(D // 16):
            rows3[2, j, pl.ds(db * 16, 16)] = z16
        return carry

    lax.fori_loop(0, K, zr, 0)
    for i in range(K // 16):
        ex3[2, pl.ds(i * 16, 16)] = z16
    plsc.subcore_barrier()

    def stage_and_fire(t, c):
        pltpu.sync_copy(sd_hbm.at[wid, c], sd3.at[t])
        pltpu.async_copy(x_hbm.at[sd3.at[t, 0]], rows3.at[t], sem_r[t])
        pltpu.async_copy(u_hbm.at[sd3.at[t, 0]], scal3.at[t, 0], sem_u[t])
        pltpu.async_copy(v_hbm.at[sd3.at[t, 1]], scal3.at[t, 1], sem_v[t])

    def drain(t):
        pltpu.make_async_copy(ex3.at[t], shared_e.at[sd3.at[t, 1]],
                              sem_e[t]).wait()
        pltpu.make_async_copy(rows3.at[t], shared_h.at[sd3.at[t, 1]],
                              sem_h[t]).wait()

    def process(t):
        pltpu.make_async_copy(u_hbm.at[sd3.at[t, 0]], scal3.at[t, 0],
                              sem_u[t]).wait()
        pltpu.make_async_copy(v_hbm.at[sd3.at[t, 1]], scal3.at[t, 1],
                              sem_v[t]).wait()
        for i in range(K // 16):
            e = (scal3[t, 0, pl.ds(i * 16, 16)]
                 + scal3[t, 1, pl.ds(i * 16, 16)])
            e = jnp.where(e >= 0.0, e, e * 0.2)
            ex3[t, pl.ds(i * 16, 16)] = jnp.exp(e)
        pltpu.async_copy(ex3.at[t], shared_e.at[sd3.at[t, 1]], sem_e[t],
                         add=True)
        pltpu.make_async_copy(x_hbm.at[sd3.at[t, 0]], rows3.at[t],
                              sem_r[t]).wait()

        def scale(kk, inner):
            sv = plsc.load_gather(ex3.at[t], [jnp.zeros((16,), jnp.int32) + kk])
            for db in range(D // 16):
                rows3[t, kk, pl.ds(db * 16, 16)] = (
                    rows3[t, kk, pl.ds(db * 16, 16)] * sv)
            return inner

        lax.fori_loop(0, K, scale, 0)
        pltpu.async_copy(rows3.at[t], shared_h.at[sd3.at[t, 1]], sem_h[t],
                         add=True)

    stage_and_fire(0, 0)
    stage_and_fire(1, 1)
    pltpu.sync_copy(sd_hbm.at[wid, 2], sd3.at[2])
    pltpu.async_copy(ex3.at[2], shared_e.at[sd3.at[2, 1]], sem_e[2], add=True)
    pltpu.async_copy(rows3.at[2], shared_h.at[sd3.at[2, 1]], sem_h[2], add=True)

    def body(i, carry):
        base = 3 * i
        process(0)
        drain(2)
        stage_and_fire(2, base + 2)
        process(1)
        drain(0)

        @pl.when(i < NBODY - 1)
        def _():
            stage_and_fire(0, base + 3)

        process(2)
        drain(1)

        @pl.when(i < NBODY - 1)
        def _():
            stage_and_fire(1, base + 4)

        return carry

    lax.fori_loop(0, NBODY, body, 0)
    drain(2)
    plsc.subcore_barrier()
    pltpu.sync_copy(shared_h.at[pl.ds(sid * EPS, EPS)],
                    hpart_hbm.at[pl.ds(cid * NPAD + sid * EPS, EPS)])
    pltpu.sync_copy(shared_e.at[pl.ds(sid * EPS, EPS)],
                    epart_hbm.at[pl.ds(cid * NPAD + sid * EPS, EPS)])


def _edge_pass(x, u, v, sd, zeros_h, zeros_pad):
    k = pl.kernel(
        _edge_kernel,
        out_type=(jax.ShapeDtypeStruct((NC * NPAD, D), jnp.float32),
                  jax.ShapeDtypeStruct((NC * NPAD,), jnp.float32)),
        mesh=_mesh(),
        compiler_params=pltpu.CompilerParams(needs_layout_passes=False),
        scratch_types=[
            pltpu.VMEM((3, 2, K), jnp.int32),
            pltpu.VMEM((3, K, D), jnp.float32),
            pltpu.VMEM((3, K), jnp.float32),
            pltpu.VMEM((3, 2, K), jnp.float32),
            pltpu.VMEM_SHARED((NPAD, D), jnp.float32),
            pltpu.VMEM_SHARED((NPAD,), jnp.float32),
        ] + [pltpu.SemaphoreType.DMA] * 15,
    )
    return k(x, u, v, sd, zeros_h, zeros_pad)


_GRID = 10
_RB = N // _GRID


def _prologue_kernel(degpair_ref, feats_ref, wc_ref, norm_ref, u_ref, v_ref):
    deg = degpair_ref[:, 0:1] + degpair_ref[:, 1:2]
    norm = lax.rsqrt(jnp.maximum(deg, 1.0))
    pq = jnp.dot(feats_ref[...], wc_ref[...], preferred_element_type=jnp.float32)
    norm_ref[...] = norm
    u_ref[...] = norm * pq[:, 0:1]
    v_ref[...] = norm * pq[:, 1:2]


def _prologue(degpair, feats, wcat):
    return pl.pallas_call(
        _prologue_kernel,
        grid=(_GRID,),
        in_specs=[
            pl.BlockSpec((_RB, 2), lambda i: (i, 0)),
            pl.BlockSpec((_RB, D), lambda i: (i, 0)),
            pl.BlockSpec((D, 2), lambda i: (0, 0)),
        ],
        out_specs=[
            pl.BlockSpec((_RB, 1), lambda i: (i, 0)),
            pl.BlockSpec((_RB, 1), lambda i: (i, 0)),
            pl.BlockSpec((_RB, 1), lambda i: (i, 0)),
        ],
        out_shape=[jax.ShapeDtypeStruct((N, 1), jnp.float32),
                   jax.ShapeDtypeStruct((NPAD, 1), jnp.float32),
                   jax.ShapeDtypeStruct((NPAD, 1), jnp.float32)],
    )(degpair, feats, wcat)


def _epilogue_kernel(hp_ref, ep_ref, y_ref, norm_ref, wc_ref, sc_ref,
                     x_ref, yo_ref, u_ref, v_ref):
    es = ep_ref[:, 0:1] + ep_ref[:, 1:2] + 1e-16
    h = (hp_ref[0] + hp_ref[1]) / es
    x_ref[...] = h
    yo_ref[...] = (y_ref[...] + h) * sc_ref[0, 0]
    norm = norm_ref[...]
    pq = jnp.dot(h, wc_ref[...], preferred_element_type=jnp.float32)
    u_ref[...] = norm * pq[:, 0:1]
    v_ref[...] = norm * pq[:, 1:2]


def _epilogue(hpart, epair, y_prev, norm, wcat, sc):
    return pl.pallas_call(
        _epilogue_kernel,
        grid=(_GRID,),
        in_specs=[
            pl.BlockSpec((NC, _RB, D), lambda i: (0, i, 0)),
            pl.BlockSpec((_RB, 2), lambda i: (i, 0)),
            pl.BlockSpec((_RB, D), lambda i: (i, 0)),
            pl.BlockSpec((_RB, 1), lambda i: (i, 0)),
            pl.BlockSpec((D, 2), lambda i: (0, 0)),
            pl.BlockSpec((1, 1), lambda i: (0, 0)),
        ],
        out_specs=[
            pl.BlockSpec((_RB, D), lambda i: (i, 0)),
            pl.BlockSpec((_RB, D), lambda i: (i, 0)),
            pl.BlockSpec((_RB, 1), lambda i: (i, 0)),
            pl.BlockSpec((_RB, 1), lambda i: (i, 0)),
        ],
        out_shape=[
            jax.ShapeDtypeStruct((N, D), jnp.float32),
            jax.ShapeDtypeStruct((N, D), jnp.float32),
            jax.ShapeDtypeStruct((NPAD, 1), jnp.float32),
            jax.ShapeDtypeStruct((NPAD, 1), jnp.float32),
        ],
    )(hpart, epair, y_prev, norm, wcat, sc)


def kernel(feats, edge_index, order, W_att):
    pad_n = EPT2 - EPT
    j = jnp.arange(pad_n, dtype=jnp.int32)[None, :]
    w = jnp.arange(NW, dtype=jnp.int32)[:, None]
    fake_src = (w * 311 + j * 37) % N
    fake_dst = N + (w * pad_n + j) % (NPAD - N)
    srcp = jnp.concatenate(
        [edge_index[0].astype(jnp.int32).reshape(NW, EPT),
         jnp.broadcast_to(fake_src, (NW, pad_n))], axis=1)
    dstp = jnp.concatenate(
        [edge_index[1].astype(jnp.int32).reshape(NW, EPT),
         jnp.broadcast_to(fake_dst, (NW, pad_n))], axis=1)
    sd = jnp.stack([srcp.reshape(NW, NCHUNK, K),
                    dstp.reshape(NW, NCHUNK, K)], axis=2)
    wcat = W_att[:, 0].reshape(2, D).T
    zeros_pad = jnp.zeros((NPAD,), jnp.float32)
    zeros_h = jnp.zeros((NPAD, D), jnp.float32)

    degpart = _deg_pass(sd, zeros_pad)
    degpair = degpart.reshape(NC, NPAD)[:, :N].T
    norm, u, v = _prologue(degpair, feats, wcat)

    one = jnp.ones((1, 1), jnp.float32)
    last = (1.0 / (order + 1.0)) * one

    x = feats
    y = feats
    for t in range(4):
        hflat, eflat = _edge_pass(x, u.reshape(NPAD), v.reshape(NPAD),
                                  sd, zeros_h, zeros_pad)
        hpart = hflat.reshape(NC, NPAD, D)
        epair = eflat.reshape(NC, NPAD)[:, :N].T
        sc = last if t == 3 else one
        x, y, u, v = _epilogue(hpart, epair, y, norm, wcat, sc)
    return y
